# baseline (speedup 1.0000x reference)
import jax
import jax.numpy as jnp
from jax import lax
from jax.experimental import pallas as pl
from jax.experimental.pallas import tpu as pltpu
from jax.experimental.pallas import tpu_sc as plsc

BATCH = 4096
NUM_FIELDS = 26
EMBED_DIM = 16
FIELD_SIZE = 40000
NUM_ROWS = FIELD_SIZE * NUM_FIELDS
NUM_WORKERS = 32
B_PER_W = BATCH // NUM_WORKERS
X_PER_W = B_PER_W * NUM_FIELDS
LANES = 16
HALF = NUM_ROWS * 8
ROWS_PER_W = NUM_FIELDS * 2 * 8
OUT_PER_W = ROWS_PER_W * 128
F_STRIDE = EMBED_DIM * BATCH
RB_STRIDE = 8 * BATCH


def _emb_body(x_hbm, tab_hbm, out_hbm, x_v, idx_v, out_v, sem, sem2):
    nc = 2
    wid = lax.axis_index("s") * nc + lax.axis_index("c")

    pltpu.sync_copy(x_hbm.at[pl.ds(wid * X_PER_W, X_PER_W)], x_v)

    iota = lax.broadcasted_iota(jnp.int32, (LANES,), 0)

    def per_field(f, carry):
        foff = f * FIELD_SIZE
        for q in range(B_PER_W // LANES):
            lanes = q * LANES + iota
            xv = plsc.load_gather(x_v, [lanes * NUM_FIELDS + f])
            idx = xv + foff
            base = (
                lax.shift_right_logical(idx, 7) * 1024
                + lax.bitwise_and(idx, 127)
            )
            for rb in range(2):
                for d8 in range(8):
                    row = f * 16 + rb * 8 + d8
                    idx_v[row, pl.ds(q * LANES, LANES)] = (
                        base + (rb * HALF + d8 * 128)
                    )
        for r in range(16):
            row = f * 16 + r
            pltpu.make_async_copy(
                tab_hbm.at[idx_v.at[row]],
                out_v.at[pl.ds(row * 128, 128)],
                sem,
            ).start()
        return carry

    lax.fori_loop(0, NUM_FIELDS, per_field, None, unroll=2)

    pltpu.make_async_copy(
        tab_hbm.at[pl.ds(0, OUT_PER_W)], out_v, sem
    ).wait()

    def store_field(f, carry):
        for rb in range(2):
            pltpu.make_async_copy(
                out_v.at[pl.ds((f * 2 + rb) * 1024, 1024)],
                out_hbm.at[
                    pl.ds(f * F_STRIDE + rb * RB_STRIDE + wid * 1024, 1024)
                ],
                sem2,
            ).start()
        return carry

    lax.fori_loop(0, NUM_FIELDS, store_field, None)
    pltpu.make_async_copy(out_v, out_hbm.at[pl.ds(0, OUT_PER_W)], sem2).wait()


@jax.jit
def kernel(x, table):
    x_flat = x.reshape(-1)
    tab_flat = (
        table.T.reshape(2, 8, NUM_ROWS // 128, 128)
        .transpose(0, 2, 1, 3)
        .reshape(-1)
    )
    mesh = plsc.VectorSubcoreMesh(core_axis_name="c", subcore_axis_name="s")
    out_flat = pl.kernel(
        _emb_body,
        out_type=jax.ShapeDtypeStruct((BATCH * EMBED_DIM * NUM_FIELDS,), jnp.float32),
        mesh=mesh,
        compiler_params=pltpu.CompilerParams(
            needs_layout_passes=False,
            use_tc_tiling_on_sc=False,
            disable_bounds_checks=True,
            disable_semaphore_checks=True,
        ),
        scratch_types=[
            pltpu.VMEM((X_PER_W,), jnp.int32),
            pltpu.VMEM((ROWS_PER_W, 128), jnp.int32),
            pltpu.VMEM((OUT_PER_W,), jnp.float32),
            pltpu.SemaphoreType.DMA,
            pltpu.SemaphoreType.DMA,
        ],
    )(x_flat, tab_flat)
    return (
        out_flat.reshape(NUM_FIELDS, 2, NUM_WORKERS, 8, 128)
        .transpose(0, 1, 3, 2, 4)
        .reshape(NUM_FIELDS, EMBED_DIM, BATCH)
        .transpose(2, 1, 0)
    )


# speedup vs baseline: 1.0026x; 1.0026x over previous
import jax
import jax.numpy as jnp
from jax import lax
from jax.experimental import pallas as pl
from jax.experimental.pallas import tpu as pltpu
from jax.experimental.pallas import tpu_sc as plsc

BATCH = 4096
NUM_FIELDS = 26
EMBED_DIM = 16
FIELD_SIZE = 40000
NUM_ROWS = FIELD_SIZE * NUM_FIELDS
NUM_WORKERS = 32
B_PER_W = BATCH // NUM_WORKERS
X_PER_W = B_PER_W * NUM_FIELDS
LANES = 16
HALF = NUM_ROWS * 8
ROWS_PER_W = NUM_FIELDS * 2 * 8
OUT_PER_W = ROWS_PER_W * 128
F_STRIDE = EMBED_DIM * BATCH
RB_STRIDE = 8 * BATCH


def _emb_body(x_hbm, tab_hbm, out_hbm, x_v, idx_v, out_v, sem, sem2):
    nc = 2
    wid = lax.axis_index("s") * nc + lax.axis_index("c")

    pltpu.sync_copy(x_hbm.at[pl.ds(wid * X_PER_W, X_PER_W)], x_v)

    iota = lax.broadcasted_iota(jnp.int32, (LANES,), 0)

    def per_field(f, carry):
        foff = f * FIELD_SIZE
        for q in range(B_PER_W // LANES):
            lanes = q * LANES + iota
            xv = plsc.load_gather(x_v, [lanes * NUM_FIELDS + f])
            idx = xv + foff
            base = (
                lax.shift_right_logical(idx, 7) * 1024
                + lax.bitwise_and(idx, 127)
            )
            for rb in range(2):
                for d8 in range(8):
                    row = f * 16 + rb * 8 + d8
                    idx_v[row, pl.ds(q * LANES, LANES)] = (
                        base + (rb * HALF + d8 * 128)
                    )
        for r in range(16):
            row = f * 16 + r
            pltpu.make_async_copy(
                tab_hbm.at[idx_v.at[row]],
                out_v.at[pl.ds(row * 128, 128)],
                sem,
            ).start()
        return carry

    lax.fori_loop(0, NUM_FIELDS, per_field, None)

    pltpu.make_async_copy(
        tab_hbm.at[pl.ds(0, OUT_PER_W)], out_v, sem
    ).wait()

    def store_field(f, carry):
        for rb in range(2):
            pltpu.make_async_copy(
                out_v.at[pl.ds((f * 2 + rb) * 1024, 1024)],
                out_hbm.at[
                    pl.ds(f * F_STRIDE + rb * RB_STRIDE + wid * 1024, 1024)
                ],
                sem2,
            ).start()
        return carry

    lax.fori_loop(0, NUM_FIELDS, store_field, None)
    pltpu.make_async_copy(out_v, out_hbm.at[pl.ds(0, OUT_PER_W)], sem2).wait()


@jax.jit
def kernel(x, table):
    x_flat = x.reshape(-1)
    tab_flat = (
        table.T.reshape(2, 8, NUM_ROWS // 128, 128)
        .transpose(0, 2, 1, 3)
        .reshape(-1)
    )
    mesh = plsc.VectorSubcoreMesh(core_axis_name="c", subcore_axis_name="s")
    out_flat = pl.kernel(
        _emb_body,
        out_type=jax.ShapeDtypeStruct((BATCH * EMBED_DIM * NUM_FIELDS,), jnp.float32),
        mesh=mesh,
        compiler_params=pltpu.CompilerParams(
            needs_layout_passes=False, use_tc_tiling_on_sc=False
        ),
        scratch_types=[
            pltpu.VMEM((X_PER_W,), jnp.int32),
            pltpu.VMEM((ROWS_PER_W, 128), jnp.int32),
            pltpu.VMEM((OUT_PER_W,), jnp.float32),
            pltpu.SemaphoreType.DMA,
            pltpu.SemaphoreType.DMA,
        ],
    )(x_flat, tab_flat)
    return (
        out_flat.reshape(NUM_FIELDS, 2, NUM_WORKERS, 8, 128)
        .transpose(0, 1, 3, 2, 4)
        .reshape(NUM_FIELDS, EMBED_DIM, BATCH)
        .transpose(2, 1, 0)
    )
